# baseline (speedup 1.0000x reference)
import functools

import jax
import jax.numpy as jnp
from jax import lax
from jax.experimental import pallas as pl
from jax.experimental.pallas import tpu as pltpu
from jax.experimental.pallas import tpu_sc as plsc

N = 10000
NP = 10240
TIN = 6
NIN = 8
H = 4
K = 16
F1 = TIN * NIN
F2 = TIN * H * NIN
ALPHA = 0.2
CS_A = 32
CS_B = 16
PAIR = NP // 16
QC0 = 320
NG_A = CS_A * K // 128
NG_B = CS_B * K // 128

_SELU_L = 1.0507009873554805
_SELU_A = 1.6732632423543772

_mesh = plsc.VectorSubcoreMesh(core_axis_name="c", subcore_axis_name="s")
_cparams = pltpu.CompilerParams(use_tc_tiling_on_sc=False, needs_layout_passes=False)


def _selu(v):
    e = jnp.exp(jnp.minimum(v, 0.0))
    return jnp.where(v > 0.0, _SELU_L * v, (_SELU_L * _SELU_A) * (e - 1.0))


def _fire(table, idx_hbm, row0, idx_v, g, sem, ng):
    pltpu.sync_copy(idx_hbm.at[pl.ds(row0, ng)], idx_v)
    for jj in range(ng):
        pltpu.async_copy(table.at[idx_v.at[jj]], g.at[pl.ds(jj * 128, 128)], sem)


def _drain(table, idx_v, g, sem, ng):
    for jj in range(ng):
        pltpu.make_async_copy(
            table.at[idx_v.at[jj]], g.at[pl.ds(jj * 128, 128)], sem
        ).wait()


@functools.partial(
    pl.kernel,
    mesh=_mesh,
    out_type=jax.ShapeDtypeStruct((NP * F2,), jnp.bfloat16),
    scratch_types=[
        [pltpu.VMEM((NG_A, 128), jnp.int32) for _ in range(2)],
        [pltpu.VMEM((CS_A * K, F1), jnp.float32) for _ in range(2)],
        pltpu.VMEM((F2,), jnp.float32),
        [pltpu.VMEM((CS_A * F2,), jnp.bfloat16) for _ in range(2)],
        pltpu.VMEM((128,), jnp.float32),
        pltpu.VMEM_SHARED((NP, F1), jnp.float32),
        pltpu.VMEM_SHARED((NP * K // 128, 128), jnp.int32),
        [pltpu.SemaphoreType.DMA for _ in range(2)],
        pltpu.SemaphoreType.DMA,
    ],
    compiler_params=_cparams,
)
def _layer1(x2d_hbm, idx_hbm, swts_hbm, agg_hbm, idx_v, g, rowtmp, aggbf,
            swts_v, xsp, idxsp, gsem, outsem):
    cid = lax.axis_index("c")
    sid = lax.axis_index("s")
    start = sid * PAIR + jnp.where(cid == 0, 0, QC0)
    npair = jnp.where(cid == 0, QC0, PAIR - QC0) // (2 * CS_A)
    pltpu.sync_copy(x2d_hbm.at[pl.ds(sid * PAIR, PAIR)],
                    xsp.at[pl.ds(sid * PAIR, PAIR)])
    pltpu.sync_copy(idx_hbm.at[pl.ds(sid * (NP * K // 128 // 16), NP * K // 128 // 16)],
                    idxsp.at[pl.ds(sid * (NP * K // 128 // 16), NP * K // 128 // 16)])
    plsc.subcore_barrier()
    pltpu.sync_copy(swts_hbm, swts_v)
    w1v = [swts_v[pl.ds(h * 16, 16)] for h in range(H)]
    b1v = swts_v[pl.ds(64, 16)]
    lane = lax.broadcasted_iota(jnp.int32, (16,), 0)
    pos = []
    for h in range(H):
        row = []
        for j in range(F1 // 16):
            f = 16 * j + lane
            row.append((f >> 3) * (H * NIN) + h * NIN + (f & 7))
        pos.append(row)

    def compute(s, b):
        base = start + s * CS_A

        def node_body(cc, carry2):
            accs = [
                [jnp.full((16,), b1v[h], jnp.float32) for _ in range(3)]
                for h in range(H)
            ]
            for k in range(K):
                r = cc * K + k
                gv = [g[b][r, pl.ds(16 * j, 16)] for j in range(3)]
                for h in range(H):
                    w = w1v[h][k]
                    for j in range(3):
                        accs[h][j] = accs[h][j] + w * gv[j]
            for h in range(H):
                for j in range(3):
                    plsc.store_scatter(rowtmp, [pos[h][j]], accs[h][j])
            rs = [rowtmp[pl.ds(16 * j, 16)] for j in range(12)]
            for j in range(11, 1, -1):
                rs[j] = (1.0 - ALPHA) * rs[j] + ALPHA * rs[j - 2]
            rs = [_selu(v) for v in rs]
            for p in range(6):
                pk = plsc.pack(rs[2 * p], rs[2 * p + 1],
                               format=plsc.PackFormat.INTERLEAVED)
                aggbf[b][pl.ds(cc * F2 + 32 * p, 32)] = pk
            return carry2

        lax.fori_loop(0, CS_A, node_body, 0)
        pltpu.async_copy(aggbf[b], agg_hbm.at[pl.ds(base * F2, CS_A * F2)], outsem)

    def drain_out(b):
        pltpu.make_async_copy(
            aggbf[b], agg_hbm.at[pl.ds(0, CS_A * F2)], outsem
        ).wait()

    _fire(xsp, idxsp, start // 8, idx_v[0], g[0], gsem[0], NG_A)

    def pipe_body(i, carry):
        s = 2 * i
        _fire(xsp, idxsp, (start + (s + 1) * CS_A) // 8, idx_v[1], g[1],
              gsem[1], NG_A)
        _drain(xsp, idx_v[0], g[0], gsem[0], NG_A)

        @pl.when(i > 0)
        def _():
            drain_out(0)
            drain_out(1)

        compute(s, 0)

        @pl.when(i < npair - 1)
        def _():
            _fire(xsp, idxsp, (start + (s + 2) * CS_A) // 8, idx_v[0],
                  g[0], gsem[0], NG_A)

        _drain(xsp, idx_v[1], g[1], gsem[1], NG_A)
        compute(s + 1, 1)
        return carry

    lax.fori_loop(0, npair, pipe_body, 0)
    drain_out(0)
    drain_out(1)


@functools.partial(
    pl.kernel,
    mesh=_mesh,
    out_type=jax.ShapeDtypeStruct((10 * N * NIN,), jnp.float32),
    scratch_types=[
        [pltpu.VMEM((NG_B, 128), jnp.int32) for _ in range(2)],
        [pltpu.VMEM((CS_B * K, F2), jnp.bfloat16) for _ in range(2)],
        pltpu.VMEM((CS_B * F2,), jnp.float32),
        [pltpu.VMEM((4 * CS_B * NIN,), jnp.float32) for _ in range(2)],
        pltpu.VMEM((7504,), jnp.float32),
        pltpu.VMEM((24, 2, 16), jnp.float32),
        pltpu.VMEM((2, 16), jnp.float32),
        pltpu.VMEM((128,), jnp.float32),
        pltpu.VMEM_SHARED((NP, F2), jnp.bfloat16),
        [pltpu.SemaphoreType.DMA for _ in range(2)],
        pltpu.SemaphoreType.DMA,
    ],
    compiler_params=_cparams,
)
def _layer2(agg_hbm, idx_hbm, wsv_hbm, bsv_hbm, swts_hbm, xflat_hbm, out_hbm,
            idx_v, g2, rowbuf, youtbuf, xbounce, wsv_v, bsv_v, swts_v, aggsp,
            gsem, outsem):
    cid = lax.axis_index("c")
    sid = lax.axis_index("s")
    wid = sid * 2 + cid
    start = sid * PAIR + jnp.where(cid == 0, 0, QC0)
    npair = jnp.where(cid == 0, QC0, PAIR - QC0) // (2 * CS_B)
    xchunk = TIN * N * NIN // 32
    for off, ln in ((0, 7504), (7504, xchunk - 7504)):
        pltpu.sync_copy(xflat_hbm.at[pl.ds(wid * xchunk + off, ln)],
                        xbounce.at[pl.ds(0, ln)])
        pltpu.sync_copy(xbounce.at[pl.ds(0, ln)],
                        out_hbm.at[pl.ds(wid * xchunk + off, ln)])
    pltpu.sync_copy(agg_hbm.at[pl.ds(sid * PAIR, PAIR)],
                    aggsp.at[pl.ds(sid * PAIR, PAIR)])
    plsc.subcore_barrier()
    pltpu.sync_copy(swts_hbm, swts_v)
    pltpu.sync_copy(wsv_hbm, wsv_v)
    pltpu.sync_copy(bsv_hbm, bsv_v)
    w2v = swts_v[pl.ds(80, 16)]
    b2s = swts_v[pl.ds(96, 16)][0]
    lane = lax.broadcasted_iota(jnp.int32, (16,), 0)
    gpos = [(th // 4) * 32 + (th % 4) * 8 + (lane & 7) for th in range(24)]
    ypos = [(2 * m + (lane >> 3)) * (CS_B * NIN) + (lane & 7) for m in range(2)]

    def compute(s, b):
        base = start + s * CS_B

        def node_body(cc, carry2):
            accs = [jnp.full((16,), b2s, jnp.float32) for _ in range(12)]
            for k in range(K):
                r = cc * K + k
                w = w2v[k]
                for p in range(6):
                    v = g2[b][r, pl.ds(32 * p, 32)]
                    va, vb = plsc.unpack(v, format=plsc.PackFormat.INTERLEAVED)
                    accs[2 * p] = accs[2 * p] + w * va
                    accs[2 * p + 1] = accs[2 * p + 1] + w * vb
            for j in range(11, 1, -1):
                accs[j] = (1.0 - ALPHA) * accs[j] + ALPHA * accs[j - 2]
            accs = [_selu(a) for a in accs]
            rbase = cc * F2
            for j in range(12):
                rowbuf[pl.ds(rbase + 16 * j, 16)] = accs[j]
            avecs = [plsc.load_gather(rowbuf, [rbase + gpos[th]]) for th in range(24)]
            for mo in range(2):
                acs = bsv_v[mo, pl.ds(0, 16)]
                for th in range(24):
                    acs = acs + wsv_v[th, mo, pl.ds(0, 16)] * avecs[th]
                plsc.store_scatter(youtbuf[b], [ypos[mo] + cc * NIN], _selu(acs))
            return carry2

        lax.fori_loop(0, CS_B, node_body, 0)

        @pl.when(base + CS_B <= N)
        def _():
            for jo in range(4):
                pltpu.async_copy(
                    youtbuf[b].at[pl.ds(jo * CS_B * NIN, CS_B * NIN)],
                    out_hbm.at[pl.ds((TIN + jo) * N * NIN + base * NIN,
                                     CS_B * NIN)],
                    outsem)

    def drain_out(s, b):
        @pl.when(start + s * CS_B + CS_B <= N)
        def _():
            for jo in range(4):
                pltpu.make_async_copy(
                    youtbuf[b].at[pl.ds(jo * CS_B * NIN, CS_B * NIN)],
                    out_hbm.at[pl.ds(0, CS_B * NIN)],
                    outsem).wait()

    _fire(aggsp, idx_hbm, start // 8, idx_v[0], g2[0], gsem[0], NG_B)

    def pipe_body(i, carry):
        s = 2 * i
        _fire(aggsp, idx_hbm, (start + (s + 1) * CS_B) // 8, idx_v[1], g2[1],
              gsem[1], NG_B)
        _drain(aggsp, idx_v[0], g2[0], gsem[0], NG_B)

        @pl.when(i > 0)
        def _():
            drain_out(s - 2, 0)
            drain_out(s - 1, 1)

        compute(s, 0)

        @pl.when(i < npair - 1)
        def _():
            _fire(aggsp, idx_hbm, (start + (s + 2) * CS_B) // 8, idx_v[0],
                  g2[0], gsem[0], NG_B)

        _drain(aggsp, idx_v[1], g2[1], gsem[1], NG_B)
        compute(s + 1, 1)
        return carry

    lax.fori_loop(0, npair, pipe_body, 0)
    drain_out(2 * npair - 2, 0)
    drain_out(2 * npair - 1, 1)


def kernel(x, nearest_nodes, W1, b1, W2, b2, Ws, bs):
    x2d = x[0].transpose(1, 0, 2).reshape(N, F1)
    x2d = jnp.pad(x2d, ((0, NP - N), (0, 0)))
    idx = jnp.pad(nearest_nodes.astype(jnp.int32), ((0, NP - N), (0, 0)))
    idx = idx.reshape(NP * K // 128, 128)
    swts = jnp.zeros((128,), jnp.float32)
    swts = swts.at[0:64].set(W1.reshape(-1))
    swts = swts.at[64:68].set(b1)
    swts = swts.at[80:96].set(W2.reshape(-1))
    swts = swts.at[96].set(b2[0])
    l16 = jnp.arange(16)
    jo = 2 * jnp.arange(2)[:, None] + (l16[None, :] >= 8)
    wsv = Ws.T[:, jo]
    bsv = bs[jo]
    agg = _layer1(x2d, idx, swts).reshape(NP, F2)
    out = _layer2(agg, idx, wsv.astype(jnp.float32), bsv.astype(jnp.float32),
                  swts, x.reshape(-1))
    return out.reshape(1, 10, N, NIN)


# speedup vs baseline: 1.2878x; 1.2878x over previous
import functools

import jax
import jax.numpy as jnp
from jax import lax
from jax.experimental import pallas as pl
from jax.experimental.pallas import tpu as pltpu
from jax.experimental.pallas import tpu_sc as plsc

N = 10000
NP = 10240
TIN = 6
NIN = 8
H = 4
K = 16
F1 = TIN * NIN
F2 = TIN * H * NIN
ALPHA = 0.2
CS_A = 32
CS_B = 16
PAIR = NP // 16
QC0 = 320
NG_A = CS_A * K // 128
NG_B = CS_B * K // 128

_SELU_L = 1.0507009873554805
_SELU_A = 1.6732632423543772

_mesh = plsc.VectorSubcoreMesh(core_axis_name="c", subcore_axis_name="s")
_cparams = pltpu.CompilerParams(use_tc_tiling_on_sc=False, needs_layout_passes=False)


def _selu(v):
    e = jnp.exp(jnp.minimum(v, 0.0))
    return jnp.where(v > 0.0, _SELU_L * v, (_SELU_L * _SELU_A) * (e - 1.0))


def _fire(table, idx_hbm, row0, idx_v, g, sem, ng):
    pltpu.sync_copy(idx_hbm.at[pl.ds(row0, ng)], idx_v)
    for jj in range(ng):
        pltpu.async_copy(table.at[idx_v.at[jj]], g.at[pl.ds(jj * 128, 128)], sem)


def _drain(table, idx_v, g, sem, ng):
    for jj in range(ng):
        pltpu.make_async_copy(
            table.at[idx_v.at[jj]], g.at[pl.ds(jj * 128, 128)], sem
        ).wait()


@functools.partial(
    pl.kernel,
    mesh=_mesh,
    out_type=jax.ShapeDtypeStruct((NP * F2,), jnp.bfloat16),
    scratch_types=[
        [pltpu.VMEM((NG_A, 128), jnp.int32) for _ in range(2)],
        [pltpu.VMEM((CS_A * K, F1), jnp.float32) for _ in range(2)],
        pltpu.VMEM((F2,), jnp.float32),
        [pltpu.VMEM((CS_A * F2,), jnp.bfloat16) for _ in range(2)],
        pltpu.VMEM((128,), jnp.float32),
        pltpu.VMEM_SHARED((NP, F1), jnp.float32),
        pltpu.VMEM_SHARED((NP * K // 128, 128), jnp.int32),
        [pltpu.SemaphoreType.DMA for _ in range(2)],
        pltpu.SemaphoreType.DMA,
    ],
    compiler_params=_cparams,
)
def _layer1(x2d_hbm, idx_hbm, swts_hbm, agg_hbm, idx_v, g, rowtmp, aggbf,
            swts_v, xsp, idxsp, gsem, outsem):
    cid = lax.axis_index("c")
    sid = lax.axis_index("s")
    start = sid * PAIR + jnp.where(cid == 0, 0, QC0)
    npair = jnp.where(cid == 0, QC0, PAIR - QC0) // (2 * CS_A)
    pltpu.sync_copy(x2d_hbm.at[pl.ds(sid * PAIR, PAIR)],
                    xsp.at[pl.ds(sid * PAIR, PAIR)])
    pltpu.sync_copy(idx_hbm.at[pl.ds(sid * (NP * K // 128 // 16), NP * K // 128 // 16)],
                    idxsp.at[pl.ds(sid * (NP * K // 128 // 16), NP * K // 128 // 16)])
    plsc.subcore_barrier()
    pltpu.sync_copy(swts_hbm, swts_v)
    w1v = [swts_v[pl.ds(h * 16, 16)] for h in range(H)]
    b1v = swts_v[pl.ds(64, 16)]
    lane = lax.broadcasted_iota(jnp.int32, (16,), 0)
    pos = []
    for h in range(H):
        row = []
        for j in range(F1 // 16):
            f = 16 * j + lane
            row.append((f >> 3) * (H * NIN) + h * NIN + (f & 7))
        pos.append(row)

    def compute(s, b):
        base = start + s * CS_A

        def node_body(cc, carry2):
            accs = [
                [jnp.full((16,), b1v[h], jnp.float32) for _ in range(3)]
                for h in range(H)
            ]
            for k in range(K):
                r = cc * K + k
                gv = [g[b][r, pl.ds(16 * j, 16)] for j in range(3)]
                for h in range(H):
                    w = w1v[h][k]
                    for j in range(3):
                        accs[h][j] = accs[h][j] + w * gv[j]
            for h in range(H):
                for j in range(3):
                    plsc.store_scatter(rowtmp, [pos[h][j]], accs[h][j])
            rs = [rowtmp[pl.ds(16 * j, 16)] for j in range(12)]
            for j in range(11, 1, -1):
                rs[j] = (1.0 - ALPHA) * rs[j] + ALPHA * rs[j - 2]
            rs = [_selu(v) for v in rs]
            for p in range(6):
                pk = plsc.pack(rs[2 * p], rs[2 * p + 1],
                               format=plsc.PackFormat.INTERLEAVED)
                aggbf[b][pl.ds(cc * F2 + 32 * p, 32)] = pk
            return carry2

        lax.fori_loop(0, CS_A, node_body, 0)
        pltpu.async_copy(aggbf[b], agg_hbm.at[pl.ds(base * F2, CS_A * F2)], outsem)

    def drain_out(b):
        pltpu.make_async_copy(
            aggbf[b], agg_hbm.at[pl.ds(0, CS_A * F2)], outsem
        ).wait()

    _fire(xsp, idxsp, start // 8, idx_v[0], g[0], gsem[0], NG_A)

    def pipe_body(i, carry):
        s = 2 * i
        _fire(xsp, idxsp, (start + (s + 1) * CS_A) // 8, idx_v[1], g[1],
              gsem[1], NG_A)
        _drain(xsp, idx_v[0], g[0], gsem[0], NG_A)

        @pl.when(i > 0)
        def _():
            drain_out(0)
            drain_out(1)

        compute(s, 0)

        @pl.when(i < npair - 1)
        def _():
            _fire(xsp, idxsp, (start + (s + 2) * CS_A) // 8, idx_v[0],
                  g[0], gsem[0], NG_A)

        _drain(xsp, idx_v[1], g[1], gsem[1], NG_A)
        compute(s + 1, 1)
        return carry

    lax.fori_loop(0, npair, pipe_body, 0)
    drain_out(0)
    drain_out(1)


@functools.partial(
    pl.kernel,
    mesh=_mesh,
    out_type=jax.ShapeDtypeStruct((NP, 32), jnp.float32),
    scratch_types=[
        [pltpu.VMEM((NG_B, 128), jnp.int32) for _ in range(2)],
        [pltpu.VMEM((CS_B * K, F2), jnp.bfloat16) for _ in range(2)],
        pltpu.VMEM((CS_B * F2,), jnp.float32),
        [pltpu.VMEM((CS_B, 32), jnp.float32) for _ in range(2)],
        pltpu.VMEM((24, 2, 16), jnp.float32),
        pltpu.VMEM((2, 16), jnp.float32),
        pltpu.VMEM((128,), jnp.float32),
        pltpu.VMEM_SHARED((NP, F2), jnp.bfloat16),
        pltpu.VMEM_SHARED((NP * K // 128, 128), jnp.int32),
        [pltpu.SemaphoreType.DMA for _ in range(2)],
        pltpu.SemaphoreType.DMA,
    ],
    compiler_params=_cparams,
)
def _layer2(agg_hbm, idx_hbm, wsv_hbm, bsv_hbm, swts_hbm, yout_hbm,
            idx_v, g2, rowbuf, youtbuf, wsv_v, bsv_v, swts_v, aggsp, idxsp,
            gsem, outsem):
    cid = lax.axis_index("c")
    sid = lax.axis_index("s")
    start = sid * PAIR + jnp.where(cid == 0, 0, QC0)
    npair = jnp.where(cid == 0, QC0, PAIR - QC0) // (2 * CS_B)
    pltpu.sync_copy(agg_hbm.at[pl.ds(sid * PAIR, PAIR)],
                    aggsp.at[pl.ds(sid * PAIR, PAIR)])
    pltpu.sync_copy(idx_hbm.at[pl.ds(sid * (NP * K // 128 // 16), NP * K // 128 // 16)],
                    idxsp.at[pl.ds(sid * (NP * K // 128 // 16), NP * K // 128 // 16)])
    plsc.subcore_barrier()
    pltpu.sync_copy(swts_hbm, swts_v)
    pltpu.sync_copy(wsv_hbm, wsv_v)
    pltpu.sync_copy(bsv_hbm, bsv_v)
    w2v = swts_v[pl.ds(80, 16)]
    b2s = swts_v[pl.ds(96, 16)][0]
    lane = lax.broadcasted_iota(jnp.int32, (16,), 0)
    gpos = [(th // 4) * 32 + (th % 4) * 8 + (lane & 7) for th in range(24)]

    def compute(s, b):
        base = start + s * CS_B

        def node_body(cc, carry2):
            accs = [jnp.full((16,), b2s, jnp.float32) for _ in range(12)]
            for k in range(K):
                r = cc * K + k
                w = w2v[k]
                for p in range(6):
                    v = g2[b][r, pl.ds(32 * p, 32)]
                    va, vb = plsc.unpack(v, format=plsc.PackFormat.INTERLEAVED)
                    accs[2 * p] = accs[2 * p] + w * va
                    accs[2 * p + 1] = accs[2 * p + 1] + w * vb
            for j in range(11, 1, -1):
                accs[j] = (1.0 - ALPHA) * accs[j] + ALPHA * accs[j - 2]
            accs = [_selu(a) for a in accs]
            rbase = cc * F2
            for j in range(12):
                rowbuf[pl.ds(rbase + 16 * j, 16)] = accs[j]
            avecs = [plsc.load_gather(rowbuf, [rbase + gpos[th]]) for th in range(24)]
            for mo in range(2):
                acs = bsv_v[mo, pl.ds(0, 16)]
                for th in range(24):
                    acs = acs + wsv_v[th, mo, pl.ds(0, 16)] * avecs[th]
                youtbuf[b][cc, pl.ds(16 * mo, 16)] = _selu(acs)
            return carry2

        lax.fori_loop(0, CS_B, node_body, 0)
        pltpu.async_copy(youtbuf[b], yout_hbm.at[pl.ds(base, CS_B)], outsem)

    def drain_out(b):
        pltpu.make_async_copy(
            youtbuf[b], yout_hbm.at[pl.ds(0, CS_B)], outsem
        ).wait()

    _fire(aggsp, idxsp, start // 8, idx_v[0], g2[0], gsem[0], NG_B)

    def pipe_body(i, carry):
        s = 2 * i
        _fire(aggsp, idxsp, (start + (s + 1) * CS_B) // 8, idx_v[1], g2[1],
              gsem[1], NG_B)
        _drain(aggsp, idx_v[0], g2[0], gsem[0], NG_B)

        @pl.when(i > 0)
        def _():
            drain_out(0)
            drain_out(1)

        compute(s, 0)

        @pl.when(i < npair - 1)
        def _():
            _fire(aggsp, idxsp, (start + (s + 2) * CS_B) // 8, idx_v[0],
                  g2[0], gsem[0], NG_B)

        _drain(aggsp, idx_v[1], g2[1], gsem[1], NG_B)
        compute(s + 1, 1)
        return carry

    lax.fori_loop(0, npair, pipe_body, 0)
    drain_out(0)
    drain_out(1)


def kernel(x, nearest_nodes, W1, b1, W2, b2, Ws, bs):
    x2d = x[0].transpose(1, 0, 2).reshape(N, F1)
    x2d = jnp.pad(x2d, ((0, NP - N), (0, 0)))
    idx = jnp.pad(nearest_nodes.astype(jnp.int32), ((0, NP - N), (0, 0)))
    idx = idx.reshape(NP * K // 128, 128)
    swts = jnp.zeros((128,), jnp.float32)
    swts = swts.at[0:64].set(W1.reshape(-1))
    swts = swts.at[64:68].set(b1)
    swts = swts.at[80:96].set(W2.reshape(-1))
    swts = swts.at[96].set(b2[0])
    l16 = jnp.arange(16)
    jo = 2 * jnp.arange(2)[:, None] + (l16[None, :] >= 8)
    wsv = Ws.T[:, jo]
    bsv = bs[jo]
    agg = _layer1(x2d, idx, swts).reshape(NP, F2)
    yout = _layer2(agg, idx, wsv.astype(jnp.float32), bsv.astype(jnp.float32), swts)
    y = yout[:N].reshape(N, 4, 8).transpose(1, 0, 2)[None]
    return jnp.concatenate([x, y], axis=1)
